# baseline (speedup 1.0000x reference)
import jax
import jax.numpy as jnp
from jax import lax
from jax.experimental import pallas as pl
from jax.experimental.pallas import tpu as pltpu
from jax.experimental.pallas import tpu_sc as plsc


def _select_last_body(in_hbm, len_hbm, out_hbm, len_smem):
    pltpu.sync_copy(len_hbm, len_smem)
    for b in range(4):
        n = len_smem[b]
        row = jnp.where(n > 0, n - 1, 4095)
        pltpu.sync_copy(in_hbm.at[b, row], out_hbm.at[b])


def kernel(input_, lengths):
    B, T, D = input_.shape
    lens = lengths.astype(jnp.int32)
    mesh = plsc.ScalarSubcoreMesh(axis_name="c", num_cores=1)
    out = pl.kernel(
        _select_last_body,
        out_type=jax.ShapeDtypeStruct((B, D), input_.dtype),
        mesh=mesh,
        scratch_types=[
            pltpu.SMEM((4,), jnp.int32),
        ],
    )(input_, lens)
    return out[:, None, :]
